# baseline (speedup 1.0000x reference)
import jax
import jax.numpy as jnp
from jax.experimental import pallas as pl
from jax.experimental.pallas import tpu as pltpu
from jax.experimental.pallas import tpu_sc as plsc

D = 64
K = 8192
TM = 512
WIN = 4096


def _dist_argmin_body(xb_ref, hi_ref, lo_ref, sx_ref, sc_ref, idx_ref):
    xb = xb_ref[...].astype(jnp.bfloat16)
    mm = (
        jax.lax.dot_general(xb, hi_ref[...], (((1,), (0,)), ((), ())),
                            preferred_element_type=jnp.float32)
        + jax.lax.dot_general(xb, lo_ref[...], (((1,), (0,)), ((), ())),
                              preferred_element_type=jnp.float32)
    )
    d = (sx_ref[...] + sc_ref[...]) - 2.0 * mm

    acc_v = jnp.full((TM,), jnp.inf, dtype=jnp.bfloat16)
    acc_i = jnp.zeros((TM,), dtype=jnp.int32)
    for w in range(K // WIN):
        win = d[:, w * WIN:(w + 1) * WIN]
        wmin = jnp.min(win, axis=1)
        iota = jax.lax.broadcasted_iota(jnp.int32, win.shape, 1) + w * WIN
        widx = jnp.min(jnp.where(win == wmin[:, None], iota, K), axis=1)
        upd = wmin < acc_v.astype(jnp.float32)
        acc_v = jnp.where(upd, wmin.astype(jnp.bfloat16), acc_v)
        acc_i = jnp.where(upd, widx, acc_i)
    idx_ref[0, 0, :] = acc_i


def _distance_argmin(xb, cbt_hi, cbt_lo, sx, sc):
    n = xb.shape[0]
    grid = n // TM
    idx3 = pl.pallas_call(
        _dist_argmin_body,
        grid=(grid,),
        in_specs=[
            pl.BlockSpec((TM, D), lambda i: (i, 0)),
            pl.BlockSpec((D, K), lambda i: (0, 0)),
            pl.BlockSpec((D, K), lambda i: (0, 0)),
            pl.BlockSpec((TM, 1), lambda i: (i, 0)),
            pl.BlockSpec((1, K), lambda i: (0, 0)),
        ],
        out_specs=pl.BlockSpec((1, 1, TM), lambda i: (i, 0, 0)),
        out_shape=jax.ShapeDtypeStruct((grid, 1, TM), jnp.int32),
        compiler_params=pltpu.CompilerParams(
            dimension_semantics=("parallel",)),
    )(xb, cbt_hi, cbt_lo, sx, sc)
    return idx3.reshape(-1)


def _sc_gather(cb_padded, indices):
    num_indices = indices.shape[0]
    ind2 = indices.reshape(1, num_indices)
    window = 128
    mesh = plsc.VectorSubcoreMesh(core_axis_name="core",
                                  subcore_axis_name="subcore")

    @pl.kernel(
        out_type=jax.ShapeDtypeStruct((num_indices, 128), cb_padded.dtype),
        mesh=mesh)
    def gather_kernel(cb_hbm, i_hbm, o_hbm):
        def body(i_vmem, o_vmem):
            pltpu.sync_copy(cb_hbm.at[i_vmem.at[0]], o_vmem)

        pltpu.emit_pipeline(
            body,
            grid=(num_indices // window,),
            in_specs=[pl.BlockSpec((1, window), index_map=lambda i: (0, i))],
            out_specs=[pl.BlockSpec((window, 128), index_map=lambda i: (i, 0))],
            core_axis_name=("core", "subcore"),
            dimension_semantics=(pltpu.PARALLEL,),
        )(i_hbm, o_hbm)

    return gather_kernel(cb_padded, ind2)


def kernel(x, codebook):
    x_flat = x.reshape(-1, D)
    xb = x_flat
    sx = jnp.sum(x_flat ** 2, axis=1, keepdims=True)
    sc = jnp.sum(codebook ** 2, axis=1)[None, :]
    cb_hi = codebook.astype(jnp.bfloat16)
    cb_lo = (codebook - cb_hi.astype(jnp.float32)).astype(jnp.bfloat16)
    indices = _distance_argmin(xb, cb_hi.T, cb_lo.T, sx, sc)
    cb_padded = jnp.pad(codebook, ((0, 0), (0, 128 - D)))
    quantized = _sc_gather(cb_padded, indices)[:, :D].reshape(x.shape)
    return quantized, indices.reshape((x.shape[0],) + x.shape[2:])


# speedup vs baseline: 1.0186x; 1.0186x over previous
import jax
import jax.numpy as jnp
from jax.experimental import pallas as pl
from jax.experimental.pallas import tpu as pltpu
from jax.experimental.pallas import tpu_sc as plsc

D = 64
K = 8192
TM = 512
WIN = 4096


def _dist_argmin_body(xb_ref, hi_ref, lo_ref, sx_ref, sc_ref, idx_ref):
    xb = xb_ref[...]
    mm = (
        jax.lax.dot_general(xb, hi_ref[...], (((1,), (0,)), ((), ())),
                            preferred_element_type=jnp.float32)
        + jax.lax.dot_general(xb, lo_ref[...], (((1,), (0,)), ((), ())),
                              preferred_element_type=jnp.float32)
    )
    d = (sx_ref[...] + sc_ref[...]) - 2.0 * mm

    acc_v = jnp.full((TM,), jnp.inf, dtype=jnp.bfloat16)
    acc_i = jnp.zeros((TM,), dtype=jnp.int32)
    for w in range(K // WIN):
        win = d[:, w * WIN:(w + 1) * WIN]
        wmin = jnp.min(win, axis=1)
        iota = jax.lax.broadcasted_iota(jnp.int32, win.shape, 1) + w * WIN
        widx = jnp.min(jnp.where(win == wmin[:, None], iota, K), axis=1)
        upd = wmin < acc_v.astype(jnp.float32)
        acc_v = jnp.where(upd, wmin.astype(jnp.bfloat16), acc_v)
        acc_i = jnp.where(upd, widx, acc_i)
    idx_ref[0, 0, :] = acc_i


def _distance_argmin(xb, cbt_hi, cbt_lo, sx, sc):
    n = xb.shape[0]
    grid = n // TM
    idx3 = pl.pallas_call(
        _dist_argmin_body,
        grid=(grid,),
        in_specs=[
            pl.BlockSpec((TM, D), lambda i: (i, 0)),
            pl.BlockSpec((D, K), lambda i: (0, 0)),
            pl.BlockSpec((D, K), lambda i: (0, 0)),
            pl.BlockSpec((TM, 1), lambda i: (i, 0)),
            pl.BlockSpec((1, K), lambda i: (0, 0)),
        ],
        out_specs=pl.BlockSpec((1, 1, TM), lambda i: (i, 0, 0)),
        out_shape=jax.ShapeDtypeStruct((grid, 1, TM), jnp.int32),
        compiler_params=pltpu.CompilerParams(
            dimension_semantics=("parallel",)),
    )(xb, cbt_hi, cbt_lo, sx, sc)
    return idx3.reshape(-1)


def _sc_gather(cb_padded, indices):
    num_indices = indices.shape[0]
    ind2 = indices.reshape(1, num_indices)
    window = 256
    mesh = plsc.VectorSubcoreMesh(core_axis_name="core",
                                  subcore_axis_name="subcore")

    @pl.kernel(
        out_type=jax.ShapeDtypeStruct((num_indices, 128), cb_padded.dtype),
        mesh=mesh)
    def gather_kernel(cb_hbm, i_hbm, o_hbm):
        def body(i_vmem, o_vmem):
            pltpu.sync_copy(cb_hbm.at[i_vmem.at[0]], o_vmem)

        pltpu.emit_pipeline(
            body,
            grid=(num_indices // window,),
            in_specs=[pl.BlockSpec((1, window), index_map=lambda i: (0, i))],
            out_specs=[pl.BlockSpec((window, 128), index_map=lambda i: (i, 0))],
            core_axis_name=("core", "subcore"),
            dimension_semantics=(pltpu.PARALLEL,),
        )(i_hbm, o_hbm)

    return gather_kernel(cb_padded, ind2)


def kernel(x, codebook):
    x_flat = x.reshape(-1, D)
    xb = x_flat.astype(jnp.bfloat16)
    sx = jnp.sum(x_flat ** 2, axis=1, keepdims=True)
    sc = jnp.sum(codebook ** 2, axis=1)[None, :]
    cb_hi = codebook.astype(jnp.bfloat16)
    cb_lo = (codebook - cb_hi.astype(jnp.float32)).astype(jnp.bfloat16)
    indices = _distance_argmin(xb, cb_hi.T, cb_lo.T, sx, sc)
    cb_padded = jnp.pad(codebook, ((0, 0), (0, 128 - D)))
    quantized = _sc_gather(cb_padded, indices)[:, :D].reshape(x.shape)
    return quantized, indices.reshape((x.shape[0],) + x.shape[2:])


# speedup vs baseline: 1.0577x; 1.0384x over previous
import jax
import jax.numpy as jnp
from jax.experimental import pallas as pl
from jax.experimental.pallas import tpu as pltpu
from jax.experimental.pallas import tpu_sc as plsc

D = 64
K = 8192
TM = 512
WIN = 4096


def _dist_argmin_body(xb_ref, hi_ref, lo_ref, sx_ref, sc_ref, idx_ref):
    xb = xb_ref[...]
    mm2 = (
        jax.lax.dot_general(xb, hi_ref[...], (((1,), (0,)), ((), ())),
                            preferred_element_type=jnp.float32)
        + jax.lax.dot_general(xb, lo_ref[...], (((1,), (0,)), ((), ())),
                              preferred_element_type=jnp.float32)
    )
    d = (sx_ref[...] + sc_ref[...]) + mm2

    acc_v = jnp.full((TM,), jnp.inf, dtype=jnp.bfloat16)
    acc_i = jnp.zeros((TM,), dtype=jnp.int32)
    for w in range(K // WIN):
        win = d[:, w * WIN:(w + 1) * WIN]
        wmin = jnp.min(win, axis=1)
        iota = jax.lax.broadcasted_iota(jnp.int32, win.shape, 1) + w * WIN
        widx = jnp.min(jnp.where(win == wmin[:, None], iota, K), axis=1)
        upd = wmin < acc_v.astype(jnp.float32)
        acc_v = jnp.where(upd, wmin.astype(jnp.bfloat16), acc_v)
        acc_i = jnp.where(upd, widx, acc_i)
    idx_ref[0, 0, :] = acc_i


def _distance_argmin(xb, cbt_hi, cbt_lo, sx, sc):
    n = xb.shape[0]
    grid = n // TM
    idx3 = pl.pallas_call(
        _dist_argmin_body,
        grid=(grid,),
        in_specs=[
            pl.BlockSpec((TM, D), lambda i: (i, 0)),
            pl.BlockSpec((D, K), lambda i: (0, 0)),
            pl.BlockSpec((D, K), lambda i: (0, 0)),
            pl.BlockSpec((TM, 1), lambda i: (i, 0)),
            pl.BlockSpec((1, K), lambda i: (0, 0)),
        ],
        out_specs=pl.BlockSpec((1, 1, TM), lambda i: (i, 0, 0)),
        out_shape=jax.ShapeDtypeStruct((grid, 1, TM), jnp.int32),
        compiler_params=pltpu.CompilerParams(
            dimension_semantics=("parallel",)),
    )(xb, cbt_hi, cbt_lo, sx, sc)
    return idx3.reshape(-1)


def _sc_gather(cb_padded, indices):
    num_indices = indices.shape[0]
    ind2 = indices.reshape(1, num_indices)
    window = 256
    mesh = plsc.VectorSubcoreMesh(core_axis_name="core",
                                  subcore_axis_name="subcore")

    @pl.kernel(
        out_type=jax.ShapeDtypeStruct((num_indices, 128), cb_padded.dtype),
        mesh=mesh)
    def gather_kernel(cb_hbm, i_hbm, o_hbm):
        def body(i_vmem, o_vmem):
            pltpu.sync_copy(cb_hbm.at[i_vmem.at[0]], o_vmem)

        pltpu.emit_pipeline(
            body,
            grid=(num_indices // window,),
            in_specs=[pl.BlockSpec((1, window), index_map=lambda i: (0, i))],
            out_specs=[pl.BlockSpec((window, 128), index_map=lambda i: (i, 0))],
            core_axis_name=("core", "subcore"),
            dimension_semantics=(pltpu.PARALLEL,),
        )(i_hbm, o_hbm)

    return gather_kernel(cb_padded, ind2)


def kernel(x, codebook):
    x_flat = x.reshape(-1, D)
    xb = x_flat.astype(jnp.bfloat16)
    sx = jnp.sum(x_flat ** 2, axis=1, keepdims=True)
    sc = jnp.sum(codebook ** 2, axis=1)[None, :]
    cb_hi = codebook.astype(jnp.bfloat16)
    cb_lo = (codebook - cb_hi.astype(jnp.float32)).astype(jnp.bfloat16)
    cb_hi2 = (cb_hi.astype(jnp.float32) * -2.0).astype(jnp.bfloat16)
    cb_lo2 = (cb_lo.astype(jnp.float32) * -2.0).astype(jnp.bfloat16)
    indices = _distance_argmin(xb, cb_hi2.T, cb_lo2.T, sx, sc)
    cb_padded = jnp.pad(codebook, ((0, 0), (0, 128 - D)))
    quantized = _sc_gather(cb_padded, indices)[:, :D].reshape(x.shape)
    return quantized, indices.reshape((x.shape[0],) + x.shape[2:])


# speedup vs baseline: 1.0579x; 1.0002x over previous
import jax
import jax.numpy as jnp
from jax.experimental import pallas as pl
from jax.experimental.pallas import tpu as pltpu
from jax.experimental.pallas import tpu_sc as plsc

D = 64
K = 8192
TM = 512
WIN = 4096


def _dist_argmin_body(xb_ref, hi_ref, lo_ref, sx_ref, sc_ref, idx_ref):
    xb = xb_ref[...]
    mm2 = (
        jax.lax.dot_general(xb, hi_ref[...], (((1,), (0,)), ((), ())),
                            preferred_element_type=jnp.float32)
        + jax.lax.dot_general(xb, lo_ref[...], (((1,), (0,)), ((), ())),
                              preferred_element_type=jnp.float32)
    )
    d = (sx_ref[...] + sc_ref[...]) + mm2

    acc_v = jnp.full((TM,), jnp.inf, dtype=jnp.bfloat16)
    acc_i = jnp.zeros((TM,), dtype=jnp.int32)
    for w in range(K // WIN):
        win = d[:, w * WIN:(w + 1) * WIN]
        wmin = jnp.min(win, axis=1)
        iota = jax.lax.broadcasted_iota(jnp.int32, win.shape, 1)
        widx = jnp.min(jnp.where(win == wmin[:, None], iota, WIN),
                       axis=1) + w * WIN
        upd = wmin < acc_v.astype(jnp.float32)
        acc_v = jnp.where(upd, wmin.astype(jnp.bfloat16), acc_v)
        acc_i = jnp.where(upd, widx, acc_i)
    idx_ref[0, 0, :] = acc_i


def _distance_argmin(xb, cbt_hi, cbt_lo, sx, sc):
    n = xb.shape[0]
    grid = n // TM
    idx3 = pl.pallas_call(
        _dist_argmin_body,
        grid=(grid,),
        in_specs=[
            pl.BlockSpec((TM, D), lambda i: (i, 0)),
            pl.BlockSpec((D, K), lambda i: (0, 0)),
            pl.BlockSpec((D, K), lambda i: (0, 0)),
            pl.BlockSpec((TM, 1), lambda i: (i, 0)),
            pl.BlockSpec((1, K), lambda i: (0, 0)),
        ],
        out_specs=pl.BlockSpec((1, 1, TM), lambda i: (i, 0, 0)),
        out_shape=jax.ShapeDtypeStruct((grid, 1, TM), jnp.int32),
        compiler_params=pltpu.CompilerParams(
            dimension_semantics=("parallel",)),
    )(xb, cbt_hi, cbt_lo, sx, sc)
    return idx3.reshape(-1)


def _sc_gather(cb_padded, indices):
    num_indices = indices.shape[0]
    ind2 = indices.reshape(1, num_indices)
    window = 256
    mesh = plsc.VectorSubcoreMesh(core_axis_name="core",
                                  subcore_axis_name="subcore")

    @pl.kernel(
        out_type=jax.ShapeDtypeStruct((num_indices, 128), cb_padded.dtype),
        mesh=mesh)
    def gather_kernel(cb_hbm, i_hbm, o_hbm):
        def body(i_vmem, o_vmem):
            pltpu.sync_copy(cb_hbm.at[i_vmem.at[0]], o_vmem)

        pltpu.emit_pipeline(
            body,
            grid=(num_indices // window,),
            in_specs=[pl.BlockSpec((1, window), index_map=lambda i: (0, i))],
            out_specs=[pl.BlockSpec((window, 128), index_map=lambda i: (i, 0))],
            core_axis_name=("core", "subcore"),
            dimension_semantics=(pltpu.PARALLEL,),
        )(i_hbm, o_hbm)

    return gather_kernel(cb_padded, ind2)


def kernel(x, codebook):
    x_flat = x.reshape(-1, D)
    xb = x_flat.astype(jnp.bfloat16)
    sx = jnp.sum(x_flat ** 2, axis=1, keepdims=True)
    sc = jnp.sum(codebook ** 2, axis=1)[None, :]
    cb_hi = codebook.astype(jnp.bfloat16)
    cb_lo = (codebook - cb_hi.astype(jnp.float32)).astype(jnp.bfloat16)
    cb_hi2 = (cb_hi.astype(jnp.float32) * -2.0).astype(jnp.bfloat16)
    cb_lo2 = (cb_lo.astype(jnp.float32) * -2.0).astype(jnp.bfloat16)
    indices = _distance_argmin(xb, cb_hi2.T, cb_lo2.T, sx, sc)
    cb_padded = jnp.pad(codebook, ((0, 0), (0, 128 - D)))
    quantized = _sc_gather(cb_padded, indices)[:, :D].reshape(x.shape)
    return quantized, indices.reshape((x.shape[0],) + x.shape[2:])
